# baseline (speedup 1.0000x reference)
import functools

import jax
import jax.numpy as jnp
from jax import lax
from jax.experimental import pallas as pl
from jax.experimental.pallas import tpu as pltpu
from jax.experimental.pallas import tpu_sc as plsc

B = 4
N = 4096
C = 64
K = 32
RSQ = 1.0
BN = B * N
RBLK = 256
NB = N // RBLK
CAP = N + 16
OB = 64
RW = BN // 32


def _phase_a(coor_ref, fea_ref, w_ref, d_ref, tau_ref, t_ref, h_ref):
    rb = pl.program_id(1)
    coor = coor_ref[0]
    coor_r = coor_ref[0, :, pl.ds(rb * RBLK, RBLK)]
    dn = (((0,), (0,)), ((), ()))
    cross = lax.dot_general(coor_r, coor, dn, preferred_element_type=jnp.float32)
    s_full = jnp.sum(coor * coor, axis=0)
    s_r = jnp.sum(coor_r * coor_r, axis=0)
    dist = s_r[:, None] + s_full[None, :] - 2.0 * cross
    d_ref[0] = dist

    cm = jnp.min(dist.reshape(RBLK, 32, 128), axis=1)

    def _peel(_, carry):
        m = jnp.min(carry, axis=1, keepdims=True)
        return jnp.where(carry == m, jnp.inf, carry)

    cm = lax.fori_loop(0, K - 1, _peel, cm)
    tau_ref[0, 0, pl.ds(rb * RBLK, RBLK)] = jnp.min(cm, axis=1)

    fea_r = fea_ref[0, :, pl.ds(rb * RBLK, RBLK)]
    wf = w_ref[0:C, :]
    wc = w_ref[C:C + 8, :][0:3, :]
    h_blk = lax.dot_general(coor_r, wc, dn, preferred_element_type=jnp.float32)
    t_blk = lax.dot_general(fea_r, wf, dn, preferred_element_type=jnp.float32)
    t_ref[0] = t_blk + h_blk
    h_ref[0] = h_blk[:, 0:C]


def _run_phase_a(points_coor, points_fea, w_pad):
    return pl.pallas_call(
        _phase_a,
        grid=(B, NB),
        in_specs=[
            pl.BlockSpec((1, 3, N), lambda b, r: (b, 0, 0)),
            pl.BlockSpec((1, C, N), lambda b, r: (b, 0, 0)),
            pl.BlockSpec((C + 8, 128), lambda b, r: (0, 0)),
        ],
        out_specs=[
            pl.BlockSpec((1, RBLK, N), lambda b, r: (b, r, 0)),
            pl.BlockSpec((1, 1, N), lambda b, r: (b, 0, 0)),
            pl.BlockSpec((1, RBLK, 128), lambda b, r: (b, r, 0)),
            pl.BlockSpec((1, RBLK, C), lambda b, r: (b, r, 0)),
        ],
        out_shape=[
            jax.ShapeDtypeStruct((B, N, N), jnp.float32),
            jax.ShapeDtypeStruct((B, 1, N), jnp.float32),
            jax.ShapeDtypeStruct((B, N, 128), jnp.float32),
            jax.ShapeDtypeStruct((B, N, C), jnp.float32),
        ],
    )(points_coor, points_fea, w_pad)


NC = N // 16


def _sc_body(d_hbm, tau_hbm, t_hbm,
             vmax_hbm, usum_hbm, qsum_hbm,
             dbuf, taubuf, cand_d, cand_j, gidx0, gidx1,
             rows0, rows1,
             obuf_mx, obuf_sm, qbuf, sem_d0, sem_d1, sem_g0, sem_g1):
    i32 = jnp.int32
    wid = lax.axis_index("s") * 2 + lax.axis_index("c")
    base = wid * RW
    tbase = (base // N) * N
    iota16 = lax.iota(i32, 16)
    ninf = jnp.float32(-jnp.inf)

    pltpu.sync_copy(tau_hbm.at[pl.ds(base, RW)], taubuf)

    def dist_start(r, doff, sem):
        pltpu.async_copy(d_hbm.at[pl.ds(pl.multiple_of((base + r) * N, N), N)],
                         dbuf.at[pl.ds(doff, N)], sem)

    def dist_wait(r, doff, sem):
        pltpu.make_async_copy(
            d_hbm.at[pl.ds(pl.multiple_of((base + r) * N, N), N)],
            dbuf.at[pl.ds(doff, N)], sem).wait()

    def select(r, doff, gidx):
        tau_sp = plsc.load_gather(taubuf, [jnp.full((16,), r, i32)])

        def chunk_body(v, cc):
            d = dbuf[pl.ds(doff + v * 16, 16)]
            m = d <= tau_sp
            ps = plsc.cumsum(jnp.where(m, 1, 0).astype(i32))
            pos = jnp.where(m, ps + (cc - 1), 0)
            plsc.store_scatter(cand_d, [pos], d, mask=m)
            plsc.store_scatter(cand_j, [pos], v * 16 + iota16, mask=m)
            return cc + jnp.max(ps)

        cc = lax.fori_loop(0, N // 16, chunk_body, jnp.int32(0), unroll=32)
        plsc.store_scatter(cand_d, [cc + iota16], jnp.full((16,), ninf))
        nv = (cc + 15) // 16

        def near_body(v, carry):
            dmin, jmin = carry
            d = cand_d[pl.ds(v * 16, 16)]
            d = jnp.where(d == ninf, jnp.inf, d)
            j = cand_j[pl.ds(v * 16, 16)]
            better = (d < dmin) | ((d == dmin) & (j < jmin))
            return jnp.where(better, d, dmin), jnp.where(better, j, jmin)

        init = (jnp.full((16,), jnp.inf, jnp.float32),
                jnp.full((16,), N, i32))
        dmin, jmin = lax.fori_loop(0, nv, near_body, init)
        dm = jnp.min(dmin)
        jnear = jnp.min(jnp.where(dmin == dm, jmin, N))
        jnear_sp = jnp.full((16,), jnear, i32)

        def peel_body(_, carry2):
            def mx_body(v, carry):
                dmax, jmax = carry
                d = cand_d[pl.ds(v * 16, 16)]
                j = cand_j[pl.ds(v * 16, 16)]
                better = (d > dmax) | ((d == dmax) & (j > jmax))
                return jnp.where(better, d, dmax), jnp.where(better, j, jmax)

            minit = (jnp.full((16,), ninf), jnp.full((16,), -1, i32))
            dmax, jmax = lax.fori_loop(0, nv, mx_body, minit)
            dM = jnp.max(dmax)
            jM = jnp.max(jnp.where(dmax == dM, jmax, -1))

            def mark_body(v, _c):
                d = cand_d[pl.ds(v * 16, 16)]
                j = cand_j[pl.ds(v * 16, 16)]
                hit = (d == dM) & (j == jM)
                cand_d[pl.ds(v * 16, 16)] = jnp.where(hit, ninf, d)
                return _c

            return lax.fori_loop(0, nv, mark_body, carry2)

        lax.fori_loop(0, cc - K, peel_body, jnp.int32(0))

        def cmp_body(v, c2):
            d = cand_d[pl.ds(v * 16, 16)]
            j = cand_j[pl.ds(v * 16, 16)]
            keep = d != ninf
            jeff = jnp.where(d <= RSQ, j, jnear_sp) + tbase
            ps = plsc.cumsum(jnp.where(keep, 1, 0).astype(i32))
            pos = jnp.where(keep, ps + (c2 - 1), 0)
            plsc.store_scatter(gidx, [pos], jeff, mask=keep)
            return c2 + jnp.max(ps)

        lax.fori_loop(0, nv, cmp_body, jnp.int32(0))

    def reduce_row(r, rows_v, qcarry):
        orow = lax.rem(r, OB)
        new_q = []
        for g in range(4):
            v0 = rows_v[0, pl.ds(g * 16, 16)]

            def red_body(s, carry):
                mx, sm, sq = carry
                v = rows_v[s, pl.ds(g * 16, 16)]
                return (jnp.maximum(mx, v), sm + v, sq + v * v)

            mx, sm, sq = lax.fori_loop(1, K, red_body, (v0, v0, v0 * v0),
                                       unroll=True)
            obuf_mx[orow, pl.ds(g * 16, 16)] = mx
            obuf_sm[orow, pl.ds(g * 16, 16)] = sm
            new_q.append(qcarry[g] + sq)

        @pl.when(orow == OB - 1)
        def _flush():
            r0 = pl.multiple_of(base + r + 1 - OB, OB)
            pltpu.sync_copy(obuf_mx, vmax_hbm.at[pl.ds(r0, OB)])
            pltpu.sync_copy(obuf_sm, usum_hbm.at[pl.ds(r0, OB)])

        return tuple(new_q)

    dist_start(0, 0, sem_d0)

    def pair_body(p, qcarry):
        r0 = 2 * p
        r1 = r0 + 1

        dist_wait(r0, 0, sem_d0)
        dist_start(r1, N, sem_d1)
        select(r0, 0, gidx0)
        g0 = pltpu.async_copy(t_hbm.at[gidx0], rows0, sem_g0)

        dist_wait(r1, N, sem_d1)

        @pl.when(r0 + 2 < RW)
        def _pf():
            dist_start(r0 + 2, 0, sem_d0)

        select(r1, N, gidx1)
        g0.wait()
        g1 = pltpu.async_copy(t_hbm.at[gidx1], rows1, sem_g1)
        qcarry = reduce_row(r0, rows0, qcarry)
        g1.wait()
        return reduce_row(r1, rows1, qcarry)

    z = jnp.zeros((16,), jnp.float32)
    qfin = lax.fori_loop(0, RW // 2, pair_body, (z, z, z, z))
    for g in range(4):
        qbuf[pl.ds(g * 16, 16)] = qfin[g]
    pltpu.sync_copy(qbuf, qsum_hbm.at[pl.ds(pl.multiple_of(wid * C, C), C)])


def _run_phase_b(d2, tau1, t2):
    mesh = plsc.VectorSubcoreMesh(core_axis_name="c", subcore_axis_name="s")
    f = functools.partial(
        pl.kernel, mesh=mesh,
        compiler_params=pltpu.CompilerParams(needs_layout_passes=False),
        out_type=[
            jax.ShapeDtypeStruct((BN, C), jnp.float32),
            jax.ShapeDtypeStruct((BN, C), jnp.float32),
            jax.ShapeDtypeStruct((32 * C,), jnp.float32),
        ],
        scratch_types=[
            pltpu.VMEM((2 * N,), jnp.float32),
            pltpu.VMEM((RW,), jnp.float32),
            pltpu.VMEM((CAP,), jnp.float32),
            pltpu.VMEM((CAP,), jnp.int32),
            pltpu.VMEM((K,), jnp.int32),
            pltpu.VMEM((K,), jnp.int32),
            pltpu.VMEM((K, 128), jnp.float32),
            pltpu.VMEM((K, 128), jnp.float32),
            pltpu.VMEM((OB, C), jnp.float32),
            pltpu.VMEM((OB, C), jnp.float32),
            pltpu.VMEM((C,), jnp.float32),
            pltpu.SemaphoreType.DMA,
            pltpu.SemaphoreType.DMA,
            pltpu.SemaphoreType.DMA,
            pltpu.SemaphoreType.DMA,
        ],
    )(_sc_body)
    return f(d2, tau1, t2)


def _stats_body(usum_ref, h_ref, qsum_ref, st_ref):
    i = pl.program_id(0)

    @pl.when(i == 0)
    def _init():
        st_ref[...] = jnp.zeros_like(st_ref)

    u = usum_ref[...]
    h = h_ref[...]
    st_ref[0, :] += jnp.sum(u, axis=0)
    st_ref[1, :] += jnp.sum(h, axis=0)
    st_ref[2, :] += jnp.sum(h * h, axis=0)
    st_ref[3, :] += jnp.sum(u * h, axis=0)

    @pl.when(i == 0)
    def _q():
        st_ref[4, :] += jnp.sum(qsum_ref[...], axis=0)


def _run_stats(usum2, h2, qsum_p):
    blk = BN // 16
    return pl.pallas_call(
        _stats_body,
        grid=(16,),
        in_specs=[
            pl.BlockSpec((blk, C), lambda i: (i, 0)),
            pl.BlockSpec((blk, C), lambda i: (i, 0)),
            pl.BlockSpec((32, C), lambda i: (0, 0)),
        ],
        out_specs=pl.BlockSpec((8, C), lambda i: (0, 0)),
        out_shape=jax.ShapeDtypeStruct((8, C), jnp.float32),
    )(usum2, h2, qsum_p)


def _final_body(vmax_ref, h_ref, st_ref, gam_ref, bet_ref, out_ref):
    cnt = jnp.float32(B * K * N)
    su = st_ref[0, :]
    sh = st_ref[1, :]
    shh = st_ref[2, :]
    sx = st_ref[3, :]
    sq = st_ref[4, :]
    mean = (su - K * sh) / cnt
    var = (sq + K * shh - 2.0 * sx) / cnt - mean * mean
    scale = gam_ref[0, :] * lax.rsqrt(var + 1e-5)
    shift = bet_ref[0, :] - mean * scale
    z = (vmax_ref[...] - h_ref[...]) * scale[None, :] + shift[None, :]
    out_ref[0] = jnp.maximum(z, 0.0).T


def _run_final(vmax2, h2, stats, gamma, beta):
    blk = 512
    nb = BN // blk
    return pl.pallas_call(
        _final_body,
        grid=(B, N // blk),
        in_specs=[
            pl.BlockSpec((blk, C), lambda b, r: (b * (nb // B) + r, 0)),
            pl.BlockSpec((blk, C), lambda b, r: (b * (nb // B) + r, 0)),
            pl.BlockSpec((8, C), lambda b, r: (0, 0)),
            pl.BlockSpec((1, C), lambda b, r: (0, 0)),
            pl.BlockSpec((1, C), lambda b, r: (0, 0)),
        ],
        out_specs=pl.BlockSpec((1, C, blk), lambda b, r: (b, 0, r)),
        out_shape=jax.ShapeDtypeStruct((B, C, N), jnp.float32),
    )(vmax2, h2, stats, gamma, beta)


def kernel(points_coor, points_fea, W, gamma, beta):
    w_pad = jnp.zeros((C + 8, 128), jnp.float32).at[0:C + 3, 0:C].set(W)
    d3, tau3, t3, h3 = _run_phase_a(points_coor, points_fea, w_pad)
    d1 = d3.reshape(BN * N)
    tau1 = tau3.reshape(BN)
    t2 = t3.reshape(BN, 128)
    h2 = h3.reshape(BN, C)
    vmax2, usum2, qsum_p = _run_phase_b(d1, tau1, t2)
    stats = _run_stats(usum2, h2, qsum_p.reshape(32, C))
    return _run_final(vmax2, h2, stats, gamma.reshape(1, C), beta.reshape(1, C))


# speedup vs baseline: 1.0187x; 1.0187x over previous
import functools

import jax
import jax.numpy as jnp
from jax import lax
from jax.experimental import pallas as pl
from jax.experimental.pallas import tpu as pltpu
from jax.experimental.pallas import tpu_sc as plsc

B = 4
N = 4096
C = 64
K = 32
RSQ = 1.0
BN = B * N
RBLK = 256
NB = N // RBLK
CAP = N + 16
OB = 64
RW = BN // 32


def _phase_a(coor_ref, fea_ref, w_ref, d_ref, tau_ref, t_ref, h_ref):
    rb = pl.program_id(1)
    coor = coor_ref[0]
    coor_r = coor_ref[0, :, pl.ds(rb * RBLK, RBLK)]
    dn = (((0,), (0,)), ((), ()))
    cross = lax.dot_general(coor_r, coor, dn, preferred_element_type=jnp.float32)
    s_full = jnp.sum(coor * coor, axis=0)
    s_r = jnp.sum(coor_r * coor_r, axis=0)
    dist = s_r[:, None] + s_full[None, :] - 2.0 * cross
    d_ref[0] = dist

    cm = jnp.min(dist.reshape(RBLK, 32, 128), axis=1)

    def _peel(_, carry):
        m = jnp.min(carry, axis=1, keepdims=True)
        return jnp.where(carry == m, jnp.inf, carry)

    cm = lax.fori_loop(0, K - 1, _peel, cm)
    tau_ref[0, 0, pl.ds(rb * RBLK, RBLK)] = jnp.min(cm, axis=1)

    fea_r = fea_ref[0, :, pl.ds(rb * RBLK, RBLK)]
    wf = w_ref[0:C, :]
    wc = w_ref[C:C + 8, :][0:3, :]
    h_blk = lax.dot_general(coor_r, wc, dn, preferred_element_type=jnp.float32)
    t_blk = lax.dot_general(fea_r, wf, dn, preferred_element_type=jnp.float32)
    t_ref[0] = t_blk + h_blk
    h_ref[0] = h_blk[:, 0:C]


def _run_phase_a(points_coor, points_fea, w_pad):
    return pl.pallas_call(
        _phase_a,
        grid=(B, NB),
        in_specs=[
            pl.BlockSpec((1, 3, N), lambda b, r: (b, 0, 0)),
            pl.BlockSpec((1, C, N), lambda b, r: (b, 0, 0)),
            pl.BlockSpec((C + 8, 128), lambda b, r: (0, 0)),
        ],
        out_specs=[
            pl.BlockSpec((1, RBLK, N), lambda b, r: (b, r, 0)),
            pl.BlockSpec((1, 1, N), lambda b, r: (b, 0, 0)),
            pl.BlockSpec((1, RBLK, 128), lambda b, r: (b, r, 0)),
            pl.BlockSpec((1, RBLK, C), lambda b, r: (b, r, 0)),
        ],
        out_shape=[
            jax.ShapeDtypeStruct((B, N, N), jnp.float32),
            jax.ShapeDtypeStruct((B, 1, N), jnp.float32),
            jax.ShapeDtypeStruct((B, N, 128), jnp.float32),
            jax.ShapeDtypeStruct((B, N, C), jnp.float32),
        ],
    )(points_coor, points_fea, w_pad)


NC = N // 16


def _sc_body(d_hbm, tau_hbm, t_hbm,
             vmax_hbm, usum_hbm, qsum_hbm,
             dbuf, taubuf, cand_d, cand_j, gidx0, gidx1,
             rows0, rows1,
             obuf_mx, obuf_sm, qbuf, sem_d0, sem_d1, sem_g0, sem_g1):
    i32 = jnp.int32
    wid = lax.axis_index("s") * 2 + lax.axis_index("c")
    base = wid * RW
    tbase = (base // N) * N
    iota16 = lax.iota(i32, 16)
    ninf = jnp.float32(-jnp.inf)

    pltpu.sync_copy(tau_hbm.at[pl.ds(base, RW)], taubuf)

    def dist_start(r, doff, sem):
        pltpu.async_copy(d_hbm.at[pl.ds(pl.multiple_of((base + r) * N, N), N)],
                         dbuf.at[pl.ds(doff, N)], sem)

    def dist_wait(r, doff, sem):
        pltpu.make_async_copy(
            d_hbm.at[pl.ds(pl.multiple_of((base + r) * N, N), N)],
            dbuf.at[pl.ds(doff, N)], sem).wait()

    def select(r, doff, gidx):
        tau_sp = plsc.load_gather(taubuf, [jnp.full((16,), r, i32)])

        def chunk_body(v, cc):
            d = dbuf[pl.ds(doff + v * 16, 16)]
            m = d <= tau_sp
            ps = plsc.cumsum(jnp.where(m, 1, 0).astype(i32))
            pos = jnp.where(m, ps + (cc - 1), 0)
            plsc.store_scatter(cand_d, [pos], d, mask=m)
            plsc.store_scatter(cand_j, [pos], v * 16 + iota16, mask=m)
            return cc + jnp.max(ps)

        cc = lax.fori_loop(0, N // 16, chunk_body, jnp.int32(0), unroll=16)
        plsc.store_scatter(cand_d, [cc + iota16], jnp.full((16,), ninf))
        nv = (cc + 15) // 16

        def near_body(v, carry):
            dmin, jmin = carry
            d = cand_d[pl.ds(v * 16, 16)]
            d = jnp.where(d == ninf, jnp.inf, d)
            j = cand_j[pl.ds(v * 16, 16)]
            better = (d < dmin) | ((d == dmin) & (j < jmin))
            return jnp.where(better, d, dmin), jnp.where(better, j, jmin)

        init = (jnp.full((16,), jnp.inf, jnp.float32),
                jnp.full((16,), N, i32))
        dmin, jmin = lax.fori_loop(0, nv, near_body, init)
        dm = jnp.min(dmin)
        jnear = jnp.min(jnp.where(dmin == dm, jmin, N))
        jnear_sp = jnp.full((16,), jnear, i32)

        def peel_body(_, carry2):
            def mx_body(v, carry):
                dmax, jmax = carry
                d = cand_d[pl.ds(v * 16, 16)]
                j = cand_j[pl.ds(v * 16, 16)]
                better = (d > dmax) | ((d == dmax) & (j > jmax))
                return jnp.where(better, d, dmax), jnp.where(better, j, jmax)

            minit = (jnp.full((16,), ninf), jnp.full((16,), -1, i32))
            dmax, jmax = lax.fori_loop(0, nv, mx_body, minit)
            dM = jnp.max(dmax)
            jM = jnp.max(jnp.where(dmax == dM, jmax, -1))

            def mark_body(v, _c):
                d = cand_d[pl.ds(v * 16, 16)]
                j = cand_j[pl.ds(v * 16, 16)]
                hit = (d == dM) & (j == jM)
                cand_d[pl.ds(v * 16, 16)] = jnp.where(hit, ninf, d)
                return _c

            return lax.fori_loop(0, nv, mark_body, carry2)

        lax.fori_loop(0, cc - K, peel_body, jnp.int32(0))

        def cmp_body(v, c2):
            d = cand_d[pl.ds(v * 16, 16)]
            j = cand_j[pl.ds(v * 16, 16)]
            keep = d != ninf
            jeff = jnp.where(d <= RSQ, j, jnear_sp) + tbase
            ps = plsc.cumsum(jnp.where(keep, 1, 0).astype(i32))
            pos = jnp.where(keep, ps + (c2 - 1), 0)
            plsc.store_scatter(gidx, [pos], jeff, mask=keep)
            return c2 + jnp.max(ps)

        lax.fori_loop(0, nv, cmp_body, jnp.int32(0))

    def reduce_row(r, rows_v, qcarry):
        orow = lax.rem(r, OB)
        new_q = []
        for g in range(4):
            v0 = rows_v[0, pl.ds(g * 16, 16)]

            def red_body(s, carry):
                mx, sm, sq = carry
                v = rows_v[s, pl.ds(g * 16, 16)]
                return (jnp.maximum(mx, v), sm + v, sq + v * v)

            mx, sm, sq = lax.fori_loop(1, K, red_body, (v0, v0, v0 * v0),
                                       unroll=True)
            obuf_mx[orow, pl.ds(g * 16, 16)] = mx
            obuf_sm[orow, pl.ds(g * 16, 16)] = sm
            new_q.append(qcarry[g] + sq)

        @pl.when(orow == OB - 1)
        def _flush():
            r0 = pl.multiple_of(base + r + 1 - OB, OB)
            pltpu.sync_copy(obuf_mx, vmax_hbm.at[pl.ds(r0, OB)])
            pltpu.sync_copy(obuf_sm, usum_hbm.at[pl.ds(r0, OB)])

        return tuple(new_q)

    dist_start(0, 0, sem_d0)

    def pair_body(p, qcarry):
        r0 = 2 * p
        r1 = r0 + 1

        dist_wait(r0, 0, sem_d0)
        dist_start(r1, N, sem_d1)
        select(r0, 0, gidx0)
        g0 = pltpu.async_copy(t_hbm.at[gidx0], rows0, sem_g0)

        dist_wait(r1, N, sem_d1)

        @pl.when(r0 + 2 < RW)
        def _pf():
            dist_start(r0 + 2, 0, sem_d0)

        select(r1, N, gidx1)
        g0.wait()
        g1 = pltpu.async_copy(t_hbm.at[gidx1], rows1, sem_g1)
        qcarry = reduce_row(r0, rows0, qcarry)
        g1.wait()
        return reduce_row(r1, rows1, qcarry)

    z = jnp.zeros((16,), jnp.float32)
    qfin = lax.fori_loop(0, RW // 2, pair_body, (z, z, z, z))
    for g in range(4):
        qbuf[pl.ds(g * 16, 16)] = qfin[g]
    pltpu.sync_copy(qbuf, qsum_hbm.at[pl.ds(pl.multiple_of(wid * C, C), C)])


def _run_phase_b(d2, tau1, t2):
    mesh = plsc.VectorSubcoreMesh(core_axis_name="c", subcore_axis_name="s")
    f = functools.partial(
        pl.kernel, mesh=mesh,
        compiler_params=pltpu.CompilerParams(needs_layout_passes=False),
        out_type=[
            jax.ShapeDtypeStruct((BN, C), jnp.float32),
            jax.ShapeDtypeStruct((BN, C), jnp.float32),
            jax.ShapeDtypeStruct((32 * C,), jnp.float32),
        ],
        scratch_types=[
            pltpu.VMEM((2 * N,), jnp.float32),
            pltpu.VMEM((RW,), jnp.float32),
            pltpu.VMEM((CAP,), jnp.float32),
            pltpu.VMEM((CAP,), jnp.int32),
            pltpu.VMEM((K,), jnp.int32),
            pltpu.VMEM((K,), jnp.int32),
            pltpu.VMEM((K, 128), jnp.float32),
            pltpu.VMEM((K, 128), jnp.float32),
            pltpu.VMEM((OB, C), jnp.float32),
            pltpu.VMEM((OB, C), jnp.float32),
            pltpu.VMEM((C,), jnp.float32),
            pltpu.SemaphoreType.DMA,
            pltpu.SemaphoreType.DMA,
            pltpu.SemaphoreType.DMA,
            pltpu.SemaphoreType.DMA,
        ],
    )(_sc_body)
    return f(d2, tau1, t2)


def _stats_body(usum_ref, h_ref, qsum_ref, st_ref):
    i = pl.program_id(0)

    @pl.when(i == 0)
    def _init():
        st_ref[...] = jnp.zeros_like(st_ref)

    u = usum_ref[...]
    h = h_ref[...]
    st_ref[0, :] += jnp.sum(u, axis=0)
    st_ref[1, :] += jnp.sum(h, axis=0)
    st_ref[2, :] += jnp.sum(h * h, axis=0)
    st_ref[3, :] += jnp.sum(u * h, axis=0)

    @pl.when(i == 0)
    def _q():
        st_ref[4, :] += jnp.sum(qsum_ref[...], axis=0)


def _run_stats(usum2, h2, qsum_p):
    blk = BN // 16
    return pl.pallas_call(
        _stats_body,
        grid=(16,),
        in_specs=[
            pl.BlockSpec((blk, C), lambda i: (i, 0)),
            pl.BlockSpec((blk, C), lambda i: (i, 0)),
            pl.BlockSpec((32, C), lambda i: (0, 0)),
        ],
        out_specs=pl.BlockSpec((8, C), lambda i: (0, 0)),
        out_shape=jax.ShapeDtypeStruct((8, C), jnp.float32),
    )(usum2, h2, qsum_p)


def _final_body(vmax_ref, h_ref, st_ref, gam_ref, bet_ref, out_ref):
    cnt = jnp.float32(B * K * N)
    su = st_ref[0, :]
    sh = st_ref[1, :]
    shh = st_ref[2, :]
    sx = st_ref[3, :]
    sq = st_ref[4, :]
    mean = (su - K * sh) / cnt
    var = (sq + K * shh - 2.0 * sx) / cnt - mean * mean
    scale = gam_ref[0, :] * lax.rsqrt(var + 1e-5)
    shift = bet_ref[0, :] - mean * scale
    z = (vmax_ref[...] - h_ref[...]) * scale[None, :] + shift[None, :]
    out_ref[0] = jnp.maximum(z, 0.0).T


def _run_final(vmax2, h2, stats, gamma, beta):
    blk = 512
    nb = BN // blk
    return pl.pallas_call(
        _final_body,
        grid=(B, N // blk),
        in_specs=[
            pl.BlockSpec((blk, C), lambda b, r: (b * (nb // B) + r, 0)),
            pl.BlockSpec((blk, C), lambda b, r: (b * (nb // B) + r, 0)),
            pl.BlockSpec((8, C), lambda b, r: (0, 0)),
            pl.BlockSpec((1, C), lambda b, r: (0, 0)),
            pl.BlockSpec((1, C), lambda b, r: (0, 0)),
        ],
        out_specs=pl.BlockSpec((1, C, blk), lambda b, r: (b, 0, r)),
        out_shape=jax.ShapeDtypeStruct((B, C, N), jnp.float32),
    )(vmax2, h2, stats, gamma, beta)


def kernel(points_coor, points_fea, W, gamma, beta):
    w_pad = jnp.zeros((C + 8, 128), jnp.float32).at[0:C + 3, 0:C].set(W)
    d3, tau3, t3, h3 = _run_phase_a(points_coor, points_fea, w_pad)
    d1 = d3.reshape(BN * N)
    tau1 = tau3.reshape(BN)
    t2 = t3.reshape(BN, 128)
    h2 = h3.reshape(BN, C)
    vmax2, usum2, qsum_p = _run_phase_b(d1, tau1, t2)
    stats = _run_stats(usum2, h2, qsum_p.reshape(32, C))
    return _run_final(vmax2, h2, stats, gamma.reshape(1, C), beta.reshape(1, C))
